# baseline (speedup 1.0000x reference)
import jax
import jax.numpy as jnp
from jax import lax
from jax.experimental import pallas as pl
from jax.experimental.pallas import tpu as pltpu
from jax.experimental.pallas import tpu_sc as plsc

NUM_UNITS = 64
MAX_REL = 128
L = 2048
ROWS = 2 * MAX_REL + 1
TV_WORDS = ROWS * NUM_UNITS
REP_ROWS = 256
REP_WORDS = REP_ROWS * NUM_UNITS
E_PAD_ROWS = 4096
E_WORDS = E_PAD_ROWS * NUM_UNITS
ROW_WORDS = L * NUM_UNITS
Q_PER_TILE = L // 32
OUT_WORDS = L * ROW_WORDS
FIRE = 8


def _sc_body(start_hbm, table_hbm, out_hbm, tv, rep, sv, e_sh, sem, bsem):
    c = lax.axis_index("c")
    s = lax.axis_index("s")

    pltpu.sync_copy(start_hbm, sv)
    start = sv[...][0]

    @pl.when(s == 0)
    def _build_e():
        pltpu.sync_copy(table_hbm, tv)
        for row, rng in ((0, range(8)), (2 * MAX_REL, range(8, 16))):
            base = row * NUM_UNITS
            v0 = tv[pl.ds(base, 16)]
            v1 = tv[pl.ds(base + 16, 16)]
            v2 = tv[pl.ds(base + 32, 16)]
            v3 = tv[pl.ds(base + 48, 16)]

            def body(r, _, v0=v0, v1=v1, v2=v2, v3=v3):
                o = r * NUM_UNITS
                rep[pl.ds(o, 16)] = v0
                rep[pl.ds(o + 16, 16)] = v1
                rep[pl.ds(o + 32, 16)] = v2
                rep[pl.ds(o + 48, 16)] = v3
                return 0

            lax.fori_loop(0, REP_ROWS, body, 0)
            for i in rng:
                pltpu.sync_copy(rep, e_sh.at[pl.ds(i * REP_WORDS, REP_WORDS)])
        off = pl.multiple_of(start * NUM_UNITS, NUM_UNITS)
        pltpu.sync_copy(tv, e_sh.at[pl.ds(off, TV_WORDS)])

    plsc.subcore_barrier()

    q_base = c * (L // 2) + s * Q_PER_TILE
    for chunk in range(0, Q_PER_TILE, FIRE):
        cps = []
        for i in range(chunk, chunk + FIRE):
            q = q_base + i
            src = e_sh.at[pl.ds((L - 1 - q) * NUM_UNITS, ROW_WORDS)]
            dst = out_hbm.at[pl.ds(q * ROW_WORDS, ROW_WORDS)]
            cps.append(pltpu.async_copy(src, dst, sem))
        for cp in cps:
            cp.wait()


def _make_sc_call():
    mesh = plsc.VectorSubcoreMesh(core_axis_name="c", subcore_axis_name="s")
    return pl.kernel(
        _sc_body,
        mesh=mesh,
        out_type=jax.ShapeDtypeStruct((OUT_WORDS,), jnp.float32),
        scratch_types=[
            pltpu.VMEM((TV_WORDS,), jnp.float32),
            pltpu.VMEM((REP_WORDS,), jnp.float32),
            pltpu.VMEM((16,), jnp.int32),
            pltpu.VMEM_SHARED((E_WORDS,), jnp.float32),
            pltpu.SemaphoreType.DMA,
            pltpu.SemaphoreType.DMA,
        ],
        compiler_params=pltpu.CompilerParams(use_tc_tiling_on_sc=False),
    )


def kernel(length_q, length_k, embeddings_table):
    start = (L - 1) - MAX_REL + (length_k - length_q)
    start_arr = jnp.full((16,), start, jnp.int32)
    table_flat = embeddings_table.reshape(TV_WORDS)
    out_flat = _make_sc_call()(start_arr, table_flat)
    return out_flat.reshape(L, L, NUM_UNITS)


# speedup vs baseline: 5.0942x; 5.0942x over previous
import jax
import jax.numpy as jnp
from jax import lax
from jax.experimental import pallas as pl
from jax.experimental.pallas import tpu as pltpu
from jax.experimental.pallas import tpu_sc as plsc

NUM_UNITS = 64
MAX_REL = 128
L = 2048
TV_WORDS = (2 * MAX_REL + 1) * NUM_UNITS
W_B = 528
OFF_MAX = W_B - 128
NSHIFT = 8
Q_PER_TILE = L // 32
NBLK = W_B // 16


def _sc_body(start_hbm, table_hbm, out_hbm, tblv, bb, sv, ew, sem, bsem):
    c = lax.axis_index("c")
    w = lax.axis_index("s")

    pltpu.sync_copy(start_hbm, sv)
    start = sv[...][0]
    m = lax.rem(lax.rem(start - 136, NSHIFT) + NSHIFT, NSHIFT)
    a0 = (start - 136) - m

    pltpu.sync_copy(table_hbm, tblv)
    iota = lax.iota(jnp.int32, 16)
    for s in range(NSHIFT):
        for r in range(4):
            u = 4 * w + r
            row = bb.at[s, r]
            base_t = a0 + s - start

            def body(k, _, row=row, u=u, base_t=base_t):
                t = iota + (16 * k + base_t)
                tix = jnp.minimum(jnp.maximum(t, 0), 2 * MAX_REL)
                v = plsc.load_gather(tblv, [tix * NUM_UNITS + u])
                row[pl.ds(16 * k, 16)] = v
                return 0

            lax.fori_loop(0, NBLK, body, 0)
        pltpu.sync_copy(bb.at[s], ew.at[s, pl.ds(4 * w, 4), :])

    plsc.subcore_barrier()

    q_base = c * (L // 2) + w * Q_PER_TILE

    def qstep(i, _):
        q = q_base + i
        c0 = (L - 1) - q
        s = lax.rem(c0, NSHIFT)
        r0 = (c0 - s) - a0
        for chunk in range(4):
            cps = []
            for ut in (2 * chunk, 2 * chunk + 1):
                for kt in range(16):
                    off = jnp.minimum(jnp.maximum(r0 + 128 * kt, 0), OFF_MAX)
                    off = pl.multiple_of(off, NSHIFT)
                    src = ew.at[s, pl.ds(8 * ut, 8), pl.ds(off, 128)]
                    dst = out_hbm.at[q, ut, kt]
                    cps.append(pltpu.async_copy(src, dst, sem))
            for cp in cps:
                cp.wait()
        return 0

    lax.fori_loop(0, Q_PER_TILE, qstep, 0)


def _make_sc_call():
    mesh = plsc.VectorSubcoreMesh(core_axis_name="c", subcore_axis_name="s")
    return pl.kernel(
        _sc_body,
        mesh=mesh,
        out_type=jax.ShapeDtypeStruct((L, 8, 16, 8, 128), jnp.float32),
        scratch_types=[
            pltpu.VMEM((TV_WORDS,), jnp.float32),
            pltpu.VMEM((NSHIFT, 4, W_B), jnp.float32),
            pltpu.VMEM((16,), jnp.int32),
            pltpu.VMEM_SHARED((NSHIFT, NUM_UNITS, W_B), jnp.float32),
            pltpu.SemaphoreType.DMA,
            pltpu.SemaphoreType.DMA,
        ],
        compiler_params=pltpu.CompilerParams(
            use_tc_tiling_on_sc=False, needs_layout_passes=False),
    )


def kernel(length_q, length_k, embeddings_table):
    start = (L - 1) - MAX_REL - (length_k - length_q)
    start_arr = jnp.full((16,), start, jnp.int32)
    table_flat = embeddings_table.reshape(TV_WORDS)
    out5 = _make_sc_call()(start_arr, table_flat)
    return jnp.transpose(out5, (0, 2, 4, 1, 3)).reshape(L, L, NUM_UNITS)
